# baseline (speedup 1.0000x reference)
import functools

import jax
import jax.numpy as jnp
from jax import lax
from jax.experimental import pallas as pl
from jax.experimental.pallas import tpu as pltpu
from jax.experimental.pallas import tpu_sc as plsc

NC = 2
NS = 16
NW = NC * NS
C = 128


@functools.partial(jax.jit, static_argnums=(2, 3))
def _sc_gather(idx_flat, table, n_per_w, total_rows):
    dim = table.shape[1]
    mesh = plsc.VectorSubcoreMesh(
        core_axis_name="c", subcore_axis_name="s",
        num_cores=NC, num_subcores=NS,
    )

    K = 10
    n_groups = n_per_w // K

    @functools.partial(
        pl.kernel,
        out_type=jax.ShapeDtypeStruct((total_rows, dim), jnp.float32),
        mesh=mesh,
        scratch_types=[
            pltpu.VMEM((n_per_w * C,), jnp.int32),
            pltpu.VMEM((2 * K * C, dim), jnp.float32),
            pltpu.SemaphoreType.DMA,
            pltpu.SemaphoreType.DMA,
        ],
        compiler_params=pltpu.CompilerParams(use_tc_tiling_on_sc=False),
    )
    def k(idx_hbm, table_hbm, out_hbm, idx_v, rows_v, sem0, sem1):
        wid = lax.axis_index("s") * NC + lax.axis_index("c")
        base = wid * n_per_w * C
        pltpu.sync_copy(idx_hbm.at[pl.ds(base, n_per_w * C)], idx_v)
        sems = (sem0, sem1)

        def issue(g, b):
            @pl.loop(0, K)
            def _(j):
                pltpu.async_copy(
                    table_hbm.at[idx_v.at[pl.ds((g * K + j) * C, C)]],
                    rows_v.at[pl.ds((b * K + j) * C, C)],
                    sems[b],
                )

        issue(0, 0)
        for g in range(n_groups):
            b = g & 1
            if g + 1 < n_groups:
                issue(g + 1, 1 - b)
            pltpu.make_async_copy(
                table_hbm.at[pl.ds(0, K * C)],
                rows_v.at[pl.ds(b * K * C, K * C)],
                sems[b],
            ).wait()
            pltpu.sync_copy(
                rows_v.at[pl.ds(b * K * C, K * C)],
                out_hbm.at[pl.ds(base + g * K * C, K * C)],
            )

    return k(idx_flat, table)


def kernel(input_indices, weight):
    b, s = input_indices.shape
    total = b * s
    idx_flat = input_indices.reshape(total).astype(jnp.int32)
    n_per_w = (total // C) // NW
    out = _sc_gather(idx_flat, weight, n_per_w, total)
    return out.reshape(b, s, weight.shape[1])


# speedup vs baseline: 1.0007x; 1.0007x over previous
import functools

import jax
import jax.numpy as jnp
from jax import lax
from jax.experimental import pallas as pl
from jax.experimental.pallas import tpu as pltpu
from jax.experimental.pallas import tpu_sc as plsc

NC = 2
NS = 16
NW = NC * NS
NSLOT = 4
C = 256


@functools.partial(jax.jit, static_argnums=(2,))
def _sc_gather(idx, table, bpw):
    n = idx.shape[0]
    dim = table.shape[1]
    mesh = plsc.VectorSubcoreMesh(
        core_axis_name="c", subcore_axis_name="s",
        num_cores=NC, num_subcores=NS,
    )
    nblk = bpw // C

    @functools.partial(
        pl.kernel,
        out_type=jax.ShapeDtypeStruct((n, dim), jnp.float32),
        mesh=mesh,
        scratch_types=[
            pltpu.VMEM((bpw,), jnp.int32),
            pltpu.VMEM((NSLOT, C, dim), jnp.float32),
            pltpu.SemaphoreType.DMA,
            pltpu.SemaphoreType.DMA,
            pltpu.SemaphoreType.DMA,
            pltpu.SemaphoreType.DMA,
            pltpu.SemaphoreType.DMA,
            pltpu.SemaphoreType.DMA,
            pltpu.SemaphoreType.DMA,
            pltpu.SemaphoreType.DMA,
        ],
        compiler_params=pltpu.CompilerParams(use_tc_tiling_on_sc=False),
    )
    def k(idx_hbm, table_hbm, out_hbm, idx_v, slab,
          g0, g1, g2, g3, w0, w1, w2, w3):
        wid = lax.axis_index("s") * NC + lax.axis_index("c")
        r0 = wid * bpw
        pltpu.sync_copy(idx_hbm.at[pl.ds(r0, bpw)], idx_v)
        gsems = (g0, g1, g2, g3)
        wsems = (w0, w1, w2, w3)

        def issue(blk, p):
            pltpu.async_copy(
                table_hbm.at[idx_v.at[pl.ds(blk * C, C)]],
                slab.at[p], gsems[p],
            )

        def wait_gather(p):
            pltpu.make_async_copy(
                table_hbm.at[pl.ds(0, C)], slab.at[p], gsems[p]
            ).wait()

        def write(blk, p):
            pltpu.async_copy(
                slab.at[p], out_hbm.at[pl.ds(r0 + blk * C, C)], wsems[p]
            )

        def wait_write(p):
            pltpu.make_async_copy(
                table_hbm.at[pl.ds(0, C)], slab.at[p], wsems[p]
            ).wait()

        for j in range(NSLOT):
            issue(j, j)

        ngrp = nblk // NSLOT

        @pl.loop(0, ngrp)
        def _(g):
            base = g * NSLOT
            for j in range(NSLOT):
                blk = base + j
                wait_gather(j)
                write(blk, j)

                @pl.when(blk + NSLOT < nblk)
                def _():
                    wait_write(j)
                    issue(blk + NSLOT, j)

        for j in range(nblk - ngrp * NSLOT):
            wait_gather(j)
            write(ngrp * NSLOT + j, j)

        for j in range(NSLOT):
            wait_write(j)

    return k(idx, table)


def kernel(input_indices, weight):
    b, s = input_indices.shape
    dim = weight.shape[1]
    idx = input_indices.reshape(b * s).astype(jnp.int32)
    bpw = (b * s) // NW
    out = _sc_gather(idx, weight, bpw)
    return out.reshape(b, s, dim)
